# baseline (speedup 1.0000x reference)
import jax
import jax.numpy as jnp
from jax.experimental import pallas as pl

K = 3
N = 10000


def _tag(x, src, dst, norm, W, b, n):
    out = x @ W[0]
    h = x
    for k in range(1, K + 1):
        h = jax.ops.segment_sum(norm[:, None] * h[src], dst, num_segments=n)
        out = out + h @ W[k]
    return out + b


def kernel(x, edge_index, edge_attr, W1, b1, Wm, bm, W9, b9):
    src, dst = edge_index[0], edge_index[1]
    deg = jnp.zeros((N,), dtype=edge_attr.dtype).at[dst].add(edge_attr)
    dis = jnp.where(deg > 0, jax.lax.rsqrt(jnp.maximum(deg, 1e-12)), 0.0)
    norm = dis[src] * edge_attr * dis[dst]
    h = jax.nn.elu(_tag(x, src, dst, norm, W1, b1, N))
    for i in range(7):
        h = jax.nn.elu(_tag(h, src, dst, norm, Wm[i], bm[i], N))
    return _tag(h, src, dst, norm, W9, b9, N)


# speedup vs baseline: 2.3037x; 2.3037x over previous
import functools

import jax
import jax.numpy as jnp
from jax import lax
from jax.experimental import pallas as pl
from jax.experimental.pallas import tpu as pltpu
from jax.experimental.pallas import tpu_sc as plsc

N = 10000
E = 320000
K = 3
F_IN = 128
H = 512
C = 40

EB = 128
R = 2560
RT = R // 16
SB = 8
NREG = 624

_MESH = dict(core_axis_name="c", subcore_axis_name="s",
             num_cores=2, num_subcores=16)


def _make_hop(nc):
    ncc = max(nc // 2, 1)
    mesh = plsc.VectorSubcoreMesh(**_MESH)

    @functools.partial(
        pl.kernel,
        out_type=jax.ShapeDtypeStruct((nc * N, 128), jnp.float32),
        mesh=mesh,
        scratch_types=dict(
            acc=pltpu.VMEM_SHARED((N, 128), jnp.float32),
            rowbuf=pltpu.VMEM((2, EB, 128), jnp.float32),
            srcb=pltpu.VMEM((SB, EB), jnp.int32),
            dstb=pltpu.VMEM((SB, EB), jnp.int32),
            nrmb=pltpu.VMEM((SB, EB), jnp.float32),
            gsem=pltpu.SemaphoreType.DMA((2,)),
            ssem=pltpu.SemaphoreType.DMA((2,)),
        ),
    )
    def hop(h_hbm, srcs_hbm, dst_hbm, nrm_hbm, out_hbm,
            acc, rowbuf, srcb, dstb, nrmb, gsem, ssem):
        c = lax.axis_index("c")
        tid = lax.axis_index("s")
        rbase = tid * RT
        nbase = tid * NREG

        def process(ch):
            zeros = jnp.zeros((16,), jnp.float32)
            for e in range(40):
                for k in range(8):
                    rowbuf[0, e, pl.ds(k * 16, 16)] = zeros
            for k in range(16):
                pltpu.sync_copy(rowbuf.at[0, pl.ds(0, 40)],
                                acc.at[pl.ds(nbase + k * 40, 40)])
            plsc.subcore_barrier()

            @pl.loop(0, RT, step=SB)
            def _super(o):
                pltpu.sync_copy(srcs_hbm.at[pl.ds(ch * R + rbase + o, SB)], srcb)
                pltpu.sync_copy(dst_hbm.at[pl.ds(rbase + o, SB)], dstb)
                pltpu.sync_copy(nrm_hbm.at[pl.ds(rbase + o, SB)], nrmb)
                pltpu.async_copy(h_hbm.at[srcb.at[0]], rowbuf.at[0], gsem.at[0])
                pltpu.async_copy(h_hbm.at[srcb.at[1]], rowbuf.at[1], gsem.at[1])

                @pl.loop(0, SB, step=2)
                def _batches(i):
                    for b in range(2):
                        ib = i + b
                        pltpu.make_async_copy(h_hbm.at[srcb.at[ib]],
                                              rowbuf.at[b], gsem.at[b]).wait()
                        for g in range(EB // 16):
                            nv = nrmb[ib, pl.ds(g * 16, 16)]
                            for j in range(16):
                                ns = jnp.take(nv, jnp.full((16,), j, jnp.int32))
                                e = g * 16 + j
                                for k in range(8):
                                    rowbuf[b, e, pl.ds(k * 16, 16)] = (
                                        rowbuf[b, e, pl.ds(k * 16, 16)] * ns)
                        pltpu.async_copy(rowbuf.at[b], acc.at[dstb.at[ib]],
                                         ssem.at[b], add=True)
                        pltpu.make_async_copy(rowbuf.at[b], acc.at[dstb.at[ib]],
                                              ssem.at[b]).wait()

                        @pl.when(ib + 2 < SB)
                        def _():
                            pltpu.async_copy(h_hbm.at[srcb.at[ib + 2]],
                                             rowbuf.at[b], gsem.at[b])

            plsc.subcore_barrier()
            pltpu.sync_copy(acc.at[pl.ds(nbase, 640)],
                            out_hbm.at[pl.ds(ch * N + nbase, 640)])
            plsc.subcore_barrier()

        if nc == 1:
            @pl.when(c == 0)
            def _():
                process(jnp.int32(0))
        else:
            @pl.loop(0, ncc)
            def _chunks(cc):
                process(c * ncc + cc)

    return hop


def _make_norm():
    mesh = plsc.VectorSubcoreMesh(**_MESH)
    RT32 = R // 32

    @functools.partial(
        pl.kernel,
        out_type=jax.ShapeDtypeStruct((R, EB), jnp.float32),
        mesh=mesh,
        compiler_params=pltpu.CompilerParams(needs_layout_passes=False),
        scratch_types=dict(
            disv=pltpu.VMEM((N,), jnp.float32),
            srcv=pltpu.VMEM((RT32, EB), jnp.int32),
            dstv=pltpu.VMEM((RT32, EB), jnp.int32),
            eav=pltpu.VMEM((RT32, EB), jnp.float32),
            outv=pltpu.VMEM((RT32, EB), jnp.float32),
        ),
    )
    def norm_k(dis_hbm, src_hbm, dst_hbm, ea_hbm, nrm_hbm,
               disv, srcv, dstv, eav, outv):
        c = lax.axis_index("c")
        tid = lax.axis_index("s")
        gtid = c * 16 + tid
        rb = gtid * RT32
        pltpu.sync_copy(dis_hbm, disv)
        pltpu.sync_copy(src_hbm.at[pl.ds(rb, RT32)], srcv)
        pltpu.sync_copy(dst_hbm.at[pl.ds(rb, RT32)], dstv)
        pltpu.sync_copy(ea_hbm.at[pl.ds(rb, RT32)], eav)

        @pl.loop(0, RT32)
        def _rows(r):
            for g in range(EB // 16):
                sl = pl.ds(g * 16, 16)
                s16 = srcv[r, sl]
                d16 = dstv[r, sl]
                a16 = eav[r, sl]
                nv = (plsc.load_gather(disv, [s16]) * a16
                      * plsc.load_gather(disv, [d16]))
                outv[r, sl] = nv

        pltpu.sync_copy(outv, nrm_hbm.at[pl.ds(rb, RT32)])

    return norm_k


def _dis_tc(deg):
    def body(d_ref, o_ref):
        d = d_ref[...]
        o_ref[...] = jnp.where(d > 0, lax.rsqrt(jnp.maximum(d, 1e-12)), 0.0)

    return pl.pallas_call(
        body,
        out_shape=jax.ShapeDtypeStruct((N, 128), jnp.float32),
        grid=(5,),
        in_specs=[pl.BlockSpec((2000, 128), lambda i: (i, 0))],
        out_specs=pl.BlockSpec((2000, 128), lambda i: (i, 0)),
    )(deg)


def _layer_tc(hs, W, b, nc_in, nc_out, apply_elu):
    Ho = 128 * nc_out
    BR = 1000

    def body(h0, h1, h2, h3, w_ref, b_ref, o_ref):
        hrefs = (h0, h1, h2, h3)
        acc = jnp.zeros((BR, Ho), jnp.float32)
        for t in range(4):
            for cc in range(nc_in):
                acc = acc + lax.dot_general(
                    hrefs[t][cc], w_ref[t, cc],
                    (((1,), (0,)), ((), ())),
                    precision=lax.Precision.HIGHEST,
                    preferred_element_type=jnp.float32)
        z = acc + b_ref[0][None, :]
        if apply_elu:
            z = jnp.where(z > 0, z, jnp.exp(jnp.minimum(z, 0.0)) - 1.0)
        for co in range(nc_out):
            o_ref[co] = z[:, co * 128:(co + 1) * 128]

    hspec = pl.BlockSpec((nc_in, BR, 128), lambda i: (0, i, 0))
    return pl.pallas_call(
        body,
        out_shape=jax.ShapeDtypeStruct((nc_out, N, 128), jnp.float32),
        grid=(N // BR,),
        in_specs=[hspec, hspec, hspec, hspec,
                  pl.BlockSpec((4, nc_in, 128, Ho), lambda i: (0, 0, 0, 0)),
                  pl.BlockSpec((1, Ho), lambda i: (0, 0))],
        out_specs=pl.BlockSpec((nc_out, BR, 128), lambda i: (0, i, 0)),
    )(*hs, W, b)


def kernel(x, edge_index, edge_attr, W1, b1, Wm, bm, W9, b9):
    src = edge_index[0]
    dst = edge_index[1]
    pad = R * EB - E
    src_p = jnp.concatenate([src, jnp.zeros((pad,), jnp.int32)])
    dst_p = jnp.concatenate([dst, jnp.zeros((pad,), jnp.int32)]).reshape(R, EB)
    ea_p = jnp.concatenate([edge_attr, jnp.zeros((pad,), jnp.float32)]
                           ).reshape(R, EB)
    src1 = src_p.reshape(R, EB)
    src4 = (src_p[None, :]
            + (jnp.arange(4, dtype=jnp.int32) * N)[:, None]).reshape(4 * R, EB)

    hop1 = _make_hop(1)
    hop4 = _make_hop(4)
    norm_k = _make_norm()

    ones = jnp.ones((N, 128), jnp.float32)
    deg = hop1(ones, src1, dst_p, ea_p)
    dis = _dis_tc(deg)
    dis_col = lax.slice(dis, (0, 0), (N, 1)).reshape(N)
    nrm = norm_k(dis_col, src1, dst_p, ea_p)

    h0 = x
    h1 = hop1(h0, src1, dst_p, nrm)
    h2 = hop1(h1, src1, dst_p, nrm)
    h3 = hop1(h2, src1, dst_p, nrm)
    W1r = W1.reshape(4, 1, 128, H)
    h = _layer_tc([h0.reshape(1, N, 128), h1.reshape(1, N, 128),
                   h2.reshape(1, N, 128), h3.reshape(1, N, 128)],
                  W1r, b1.reshape(1, H), 1, 4, True)

    for i in range(7):
        f0 = h.reshape(4 * N, 128)
        f1 = hop4(f0, src4, dst_p, nrm)
        f2 = hop4(f1, src4, dst_p, nrm)
        f3 = hop4(f2, src4, dst_p, nrm)
        Wr = Wm[i].reshape(4, 4, 128, H)
        h = _layer_tc([h, f1.reshape(4, N, 128), f2.reshape(4, N, 128),
                       f3.reshape(4, N, 128)],
                      Wr, bm[i].reshape(1, H), 4, 4, True)

    f0 = h.reshape(4 * N, 128)
    f1 = hop4(f0, src4, dst_p, nrm)
    f2 = hop4(f1, src4, dst_p, nrm)
    f3 = hop4(f2, src4, dst_p, nrm)
    W9p = jnp.pad(W9, ((0, 0), (0, 0), (0, 128 - C)))
    b9p = jnp.pad(b9, (0, 128 - C))
    out = _layer_tc([h, f1.reshape(4, N, 128), f2.reshape(4, N, 128),
                     f3.reshape(4, N, 128)],
                    W9p.reshape(4, 4, 128, 128), b9p.reshape(1, 128),
                    4, 1, False)
    return out.reshape(N, 128)[:, :C]
